# baseline (speedup 1.0000x reference)
import functools

import jax
import jax.numpy as jnp
from jax import lax
from jax.experimental import pallas as pl
from jax.experimental.pallas import tpu as pltpu
from jax.experimental.pallas import tpu_sc as plsc

NN = 10000
NP = 10240
NE = 160000
NG = 16
DC = 128
BM = 1024
EB = 125
NB = 80
HB = 40
CMAX = 16
NROW = NP // 16
F32 = jnp.float32


def _sc_agg(C):
    mesh = plsc.VectorSubcoreMesh(core_axis_name="c", subcore_axis_name="s")
    ncpc = (C + 1) // 2

    @functools.partial(
        pl.kernel,
        out_type=jax.ShapeDtypeStruct((C, NP, DC), F32),
        mesh=mesh,
        scratch_types=[
            pltpu.VMEM((HB, EB), jnp.int32),
            pltpu.VMEM((NB, EB), jnp.int32),
            pltpu.VMEM((EB, DC), F32),
            pltpu.VMEM((EB, DC), F32),
            pltpu.VMEM_SHARED((NP, DC), F32),
            pltpu.SemaphoreType.DMA,
            pltpu.SemaphoreType.DMA,
            pltpu.SemaphoreType.DMA,
            pltpu.SemaphoreType.DMA,
        ],
    )
    def k(u_hbm, idx16_hbm, dst_hbm, out_hbm, gidx_v, didx_v, rows0_v,
          rows1_v, acc_sh, sem0, sem1, ssem0, ssem1):
        core = lax.axis_index("c")
        s = lax.axis_index("s")
        pltpu.sync_copy(dst_hbm.at[pl.ds(s * NB, NB)], didx_v)
        for kk in range(ncpc):
            c = kk * 2 + core

            @pl.when(c < C)
            def _():
                pltpu.sync_copy(
                    u_hbm.at[pl.ds(c * NP + s * NROW, NROW)],
                    acc_sh.at[pl.ds(s * NROW, NROW)],
                )
                plsc.subcore_barrier()
                for h in range(NB // HB):
                    pltpu.sync_copy(
                        idx16_hbm.at[c, pl.ds(s * NB + h * HB, HB)], gidx_v)
                    pltpu.async_copy(u_hbm.at[gidx_v.at[0]], rows0_v, sem0)
                    pltpu.async_copy(u_hbm.at[gidx_v.at[1]], rows1_v, sem1)

                    def body(p, carry, h=h):
                        b0 = 2 * p
                        d0 = h * HB + b0
                        pltpu.make_async_copy(u_hbm.at[gidx_v.at[b0]],
                                              rows0_v, sem0).wait()
                        sc0 = pltpu.async_copy(
                            rows0_v, acc_sh.at[didx_v.at[d0]], ssem0,
                            add=True)
                        pltpu.make_async_copy(u_hbm.at[gidx_v.at[b0 + 1]],
                                              rows1_v, sem1).wait()
                        sc1 = pltpu.async_copy(
                            rows1_v, acc_sh.at[didx_v.at[d0 + 1]], ssem1,
                            add=True)
                        sc0.wait()

                        @pl.when(b0 + 2 < HB)
                        def _():
                            pltpu.async_copy(u_hbm.at[gidx_v.at[b0 + 2]],
                                             rows0_v, sem0)

                        sc1.wait()

                        @pl.when(b0 + 3 < HB)
                        def _():
                            pltpu.async_copy(u_hbm.at[gidx_v.at[b0 + 3]],
                                             rows1_v, sem1)

                        return carry

                    lax.fori_loop(0, HB // 2, body, 0)
                plsc.subcore_barrier()
                pltpu.sync_copy(
                    acc_sh.at[pl.ds(s * NROW, NROW)],
                    out_hbm.at[c, pl.ds(s * NROW, NROW)],
                )

    return k


def _sc_deg():
    mesh = plsc.VectorSubcoreMesh(core_axis_name="c", subcore_axis_name="s")

    @functools.partial(
        pl.kernel,
        out_type=jax.ShapeDtypeStruct((NP, 16), F32),
        mesh=mesh,
        scratch_types=[
            pltpu.VMEM((NB, EB), jnp.int32),
            pltpu.VMEM((EB, 16), F32),
            pltpu.VMEM_SHARED((NP, 16), F32),
        ],
    )
    def k(dst_hbm, zeros_hbm, out_hbm, didx_v, ones_v, acc_sh):
        core = lax.axis_index("c")
        s = lax.axis_index("s")

        @pl.when(core == 0)
        def _():
            pltpu.sync_copy(dst_hbm.at[pl.ds(s * NB, NB)], didx_v)
            pltpu.sync_copy(
                zeros_hbm.at[pl.ds(s * NROW, NROW)],
                acc_sh.at[pl.ds(s * NROW, NROW)],
            )
            vec = jnp.where(lax.iota(jnp.int32, 16) == 0, 1.0, 0.0).astype(F32)
            for r in range(EB):
                ones_v[r] = vec
            plsc.subcore_barrier()

            def body(b, carry):
                pltpu.sync_copy(ones_v, acc_sh.at[didx_v.at[b]], add=True)
                return carry

            lax.fori_loop(0, NB, body, 0)
            plsc.subcore_barrier()
            pltpu.sync_copy(
                acc_sh.at[pl.ds(s * NROW, NROW)],
                out_hbm.at[pl.ds(s * NROW, NROW)],
            )

    return k


def _agg_call(C, u_flat, idx16, dst2d):
    return _sc_agg(C)(u_flat, idx16, dst2d)


def _deg_call(dst2d, zeros16):
    return _sc_deg()(dst2d, zeros16)


def _rowmask(i, w, bm=BM):
    rid = lax.broadcasted_iota(jnp.int32, (bm, w), 0) + i * bm
    return rid < NN


def _bn_coef(cp):
    mu = cp[1:2] * (1.0 / NN)
    var = cp[2:3] * (1.0 / NN) - mu * mu
    alpha = cp[3:4] * lax.rsqrt(var + 1e-5)
    beta = cp[4:5] - mu * alpha
    return alpha, beta


def _pre(pre, s_blk, dinv_blk, cp):
    if pre == "dinv":
        return dinv_blk * s_blk
    if pre == "scn":
        z = jnp.maximum(dinv_blk * s_blk + cp[0:1], 0.0)
        alpha, beta = _bn_coef(cp)
        return z * alpha + beta
    alpha, beta = _bn_coef(cp)
    return s_blk * alpha + beta


def _mm_rs(S, W, dinv, bo):
    C = S.shape[0]
    dout = W.shape[1]
    bm = 512
    n_i = NP // bm

    def body(s_ref, w_ref, dinv_ref, bo_ref, z_ref, st_ref, acc_ref):
        i = pl.program_id(0)
        k = pl.program_id(1)
        nk = pl.num_programs(1)

        @pl.when(k == 0)
        def _():
            acc_ref[...] = jnp.zeros((bm, dout), F32)

        lhs = dinv_ref[...] * s_ref[0]
        acc_ref[...] += jnp.dot(lhs, w_ref[...], preferred_element_type=F32)

        @pl.when(k == nk - 1)
        def _():
            z = jnp.maximum(acc_ref[...] + bo_ref[0:1], 0.0)
            z_ref[...] = z
            zm = jnp.where(_rowmask(i, dout, bm), z, 0.0)

            @pl.when(i == 0)
            def _():
                st_ref[...] = jnp.zeros((8, dout), F32)

            st_ref[1:2] += jnp.sum(zm, 0, keepdims=True)
            st_ref[2:3] += jnp.sum(zm * zm, 0, keepdims=True)

    return pl.pallas_call(
        body,
        grid=(n_i, C),
        in_specs=[
            pl.BlockSpec((1, bm, DC), lambda i, k: (k, i, 0)),
            pl.BlockSpec((DC, dout), lambda i, k: (k, 0)),
            pl.BlockSpec((bm, DC), lambda i, k: (i, 0)),
            pl.BlockSpec((8, dout), lambda i, k: (0, 0)),
        ],
        out_specs=[
            pl.BlockSpec((bm, dout), lambda i, k: (i, 0)),
            pl.BlockSpec((8, dout), lambda i, k: (0, 0)),
        ],
        out_shape=[
            jax.ShapeDtypeStruct((NP, dout), F32),
            jax.ShapeDtypeStruct((8, dout), F32),
        ],
        scratch_shapes=[pltpu.VMEM((bm, dout), F32)],
    )(S, W, dinv, bo)


def _mm_uc(lhs_in, W, dinv, cp, pre, bk):
    chunks = pre == "scn"
    K = W.shape[0]
    dout = W.shape[1]
    cout = dout // DC
    n_i = NP // BM
    n_k = K // bk

    def body(l_ref, w_ref, dinv_ref, cp_ref, u_ref, acc_ref):
        k = pl.program_id(1)
        nk = pl.num_programs(1)

        @pl.when(k == 0)
        def _():
            acc_ref[...] = jnp.zeros((BM, dout), F32)

        s_blk = l_ref[0] if chunks else l_ref[...]
        lhs = _pre(pre, s_blk, dinv_ref[...], cp_ref[...])
        acc_ref[...] += jnp.dot(lhs, w_ref[...], preferred_element_type=F32)

        @pl.when(k == nk - 1)
        def _():
            for cc in range(cout):
                u_ref[cc] = dinv_ref[...] * acc_ref[:, cc * DC:(cc + 1) * DC]

    lhs_spec = (
        pl.BlockSpec((1, BM, DC), lambda i, k: (k, i, 0))
        if chunks else pl.BlockSpec((BM, bk), lambda i, k: (i, k))
    )
    return pl.pallas_call(
        body,
        grid=(n_i, n_k),
        in_specs=[
            lhs_spec,
            pl.BlockSpec((bk, dout), lambda i, k: (k, 0)),
            pl.BlockSpec((BM, DC), lambda i, k: (i, 0)),
            pl.BlockSpec((8, bk), lambda i, k: (0, k)),
        ],
        out_specs=[pl.BlockSpec((cout, BM, DC), lambda i, k: (0, i, 0))],
        out_shape=[jax.ShapeDtypeStruct((cout, NP, DC), F32)],
        scratch_shapes=[pltpu.VMEM((BM, dout), F32)],
    )(lhs_in, W, dinv, cp)[0]


def _stats(S, dinv, cp):
    C = S.shape[0]

    def body(s_ref, dinv_ref, cp_ref, st_ref):
        i = pl.program_id(1)
        z = jnp.maximum(dinv_ref[...] * s_ref[0] + cp_ref[0:1], 0.0)
        zm = jnp.where(_rowmask(i, DC), z, 0.0)

        @pl.when(i == 0)
        def _():
            st_ref[...] = jnp.zeros((8, DC), F32)

        st_ref[1:2] += jnp.sum(zm, 0, keepdims=True)
        st_ref[2:3] += jnp.sum(zm * zm, 0, keepdims=True)

    return pl.pallas_call(
        body,
        grid=(C, NP // BM),
        in_specs=[
            pl.BlockSpec((1, BM, DC), lambda c, i: (c, i, 0)),
            pl.BlockSpec((BM, DC), lambda c, i: (i, 0)),
            pl.BlockSpec((8, DC), lambda c, i: (0, c)),
        ],
        out_specs=pl.BlockSpec((8, DC), lambda c, i: (0, c)),
        out_shape=jax.ShapeDtypeStruct((8, C * DC), F32),
    )(S, dinv, cp)


def _chunk_x(xp, dinv):
    C = xp.shape[1] // DC

    def body(x_ref, dinv_ref, u_ref):
        u_ref[0] = dinv_ref[...] * x_ref[...]

    return pl.pallas_call(
        body,
        grid=(C, NP // BM),
        in_specs=[
            pl.BlockSpec((BM, DC), lambda c, i: (i, c)),
            pl.BlockSpec((BM, DC), lambda c, i: (i, 0)),
        ],
        out_specs=pl.BlockSpec((1, BM, DC), lambda c, i: (c, i, 0)),
        out_shape=jax.ShapeDtypeStruct((C, NP, DC), F32),
    )(xp, dinv)


def _chunk_u5(S4, dinv, cp):
    C = S4.shape[0]

    def body(s_ref, dinv_ref, cp_ref, u_ref):
        h = _pre("scn", s_ref[0], dinv_ref[...], cp_ref[...])
        u_ref[0] = dinv_ref[...] * h

    return pl.pallas_call(
        body,
        grid=(C, NP // BM),
        in_specs=[
            pl.BlockSpec((1, BM, DC), lambda c, i: (c, i, 0)),
            pl.BlockSpec((BM, DC), lambda c, i: (i, 0)),
            pl.BlockSpec((8, DC), lambda c, i: (0, c)),
        ],
        out_specs=pl.BlockSpec((1, BM, DC), lambda c, i: (c, i, 0)),
        out_shape=jax.ShapeDtypeStruct((C, NP, DC), F32),
    )(S4, dinv, cp)


def _dinvk(deg16):
    def body(d_ref, o_ref):
        d = d_ref[:, 0:1] + 1.0
        m = _rowmask(pl.program_id(0), 1)
        v = jnp.where(m, lax.rsqrt(d), 0.0)
        o_ref[...] = jnp.broadcast_to(v, (BM, DC))

    return pl.pallas_call(
        body,
        grid=(NP // BM,),
        in_specs=[pl.BlockSpec((BM, 16), lambda i: (i, 0))],
        out_specs=pl.BlockSpec((BM, DC), lambda i: (i, 0)),
        out_shape=jax.ShapeDtypeStruct((NP, DC), F32),
    )(deg16)


def _final(S7, dinv, batchb, b7p):
    def body(s_ref, dinv_ref, b_ref, b7_ref, o_ref):
        i = pl.program_id(0)
        m = _rowmask(i, DC)
        w = dinv_ref[...] * s_ref[0] + jnp.where(m, b7_ref[0:1], 0.0)
        gcol = lax.broadcasted_iota(jnp.int32, (BM, DC), 1)
        oh = (b_ref[...] == gcol).astype(F32)
        p = lax.dot_general(oh, w, (((0,), (0,)), ((), ())),
                            preferred_element_type=F32)

        @pl.when(i == 0)
        def _():
            o_ref[...] = jnp.zeros((DC, DC), F32)

        o_ref[...] += p

    return pl.pallas_call(
        body,
        grid=(NP // BM,),
        in_specs=[
            pl.BlockSpec((1, BM, DC), lambda i: (0, i, 0)),
            pl.BlockSpec((BM, DC), lambda i: (i, 0)),
            pl.BlockSpec((BM, DC), lambda i: (i, 0)),
            pl.BlockSpec((8, DC), lambda i: (0, 0)),
        ],
        out_specs=pl.BlockSpec((DC, DC), lambda i: (0, 0)),
        out_shape=jax.ShapeDtypeStruct((DC, DC), F32),
    )(S7, dinv, batchb, b7p)


def _cp_pack(K, b=None, st=None, g=None, be=None):
    P = jnp.zeros((8, K), F32)
    if b is not None:
        P = P.at[0, : b.shape[0]].set(b)
    if g is not None:
        P = P.at[3, : g.shape[0]].set(g)
    if be is not None:
        P = P.at[4, : be.shape[0]].set(be)
    if st is not None:
        P = P + st
    return P


def _bo_pack(b, dout):
    return jnp.zeros((8, dout), F32).at[0, : b.shape[0]].set(b)


def kernel(x, edge_index, batch, W1, b1, W2, b2, W3, b3, W4, b4, W5, b5,
           W6, b6, W7, b7, g1, be1, g2, be2, g3, be3, g4, be4, g5, be5,
           g6, be6):
    src = edge_index[0]
    dst = edge_index[1]

    dst2d = dst.reshape(16 * NB, EB)
    offs = (jnp.arange(CMAX, dtype=jnp.int32) * NP)[:, None]
    idx16 = (src[None, :] + offs).reshape(CMAX, 16 * NB, EB)
    zeros16 = jnp.zeros((NP, 16), F32)
    batchb = jnp.broadcast_to(
        jnp.pad(batch, (0, NP - NN), constant_values=NG)[:, None], (NP, DC)
    )
    xp = jnp.pad(x, ((0, NP - NN), (0, 512 - x.shape[1])))
    W1p = jnp.pad(W1, ((0, 512 - W1.shape[0]), (0, 0)))
    W7p = jnp.pad(W7, ((0, 0), (0, DC - W7.shape[1])))
    b7p = jnp.zeros((8, DC), F32).at[0, 0].set(b7[0])

    deg16 = _deg_call(dst2d, zeros16)
    dinv = _dinvk(deg16)

    u1 = _chunk_x(xp, dinv)
    S1 = _agg_call(4, u1.reshape(4 * NP, DC), idx16, dst2d)
    z1, st1 = _mm_rs(S1, W1p, dinv, _bo_pack(b1, 4096))

    u2 = _mm_uc(z1, W2, dinv, _cp_pack(4096, st=st1, g=g1, be=be1), "bn", 512)
    S2 = _agg_call(16, u2.reshape(16 * NP, DC), idx16, dst2d)

    st2 = _stats(S2, dinv, _cp_pack(2048, b=b2))
    u3 = _mm_uc(S2, W3, dinv, _cp_pack(2048, b=b2, st=st2, g=g2, be=be2),
                "scn", DC)
    S3 = _agg_call(8, u3.reshape(8 * NP, DC), idx16, dst2d)

    st3 = _stats(S3, dinv, _cp_pack(1024, b=b3))
    u4 = _mm_uc(S3, W4, dinv, _cp_pack(1024, b=b3, st=st3, g=g3, be=be3),
                "scn", DC)
    S4 = _agg_call(8, u4.reshape(8 * NP, DC), idx16, dst2d)

    st4 = _stats(S4, dinv, _cp_pack(1024, b=b4))
    u5 = _chunk_u5(S4, dinv, _cp_pack(1024, b=b4, st=st4, g=g4, be=be4))
    S5 = _agg_call(8, u5.reshape(8 * NP, DC), idx16, dst2d)
    z5, st5 = _mm_rs(S5, W5, dinv, _bo_pack(b5, 2048))

    u6 = _mm_uc(z5, W6, dinv, _cp_pack(2048, st=st5, g=g5, be=be5), "bn", 512)
    S6 = _agg_call(2, u6.reshape(2 * NP, DC), idx16, dst2d)

    st6 = _stats(S6, dinv, _cp_pack(256, b=b6))
    u7 = _mm_uc(S6, W7p, dinv, _cp_pack(256, b=b6, st=st6, g=g6, be=be6),
                "scn", DC)
    S7 = _agg_call(1, u7.reshape(NP, DC), idx16, dst2d)

    res = _final(S7, dinv, batchb, b7p)
    return res[:NG, 0:1]


# speedup vs baseline: 1.2319x; 1.2319x over previous
import functools

import jax
import jax.numpy as jnp
from jax import lax
from jax.experimental import pallas as pl
from jax.experimental.pallas import tpu as pltpu
from jax.experimental.pallas import tpu_sc as plsc

NN = 10000
NP = 10240
NE = 160000
NG = 16
DC = 128
BM = 1024
EB = 125
NB = 80
HB = 40
CMAX = 16
NROW = NP // 16
F32 = jnp.float32


def _sc_agg(C, dc=DC):
    mesh = plsc.VectorSubcoreMesh(core_axis_name="c", subcore_axis_name="s")
    ncpc = (C + 1) // 2

    @functools.partial(
        pl.kernel,
        out_type=jax.ShapeDtypeStruct((C, NP, dc), F32),
        mesh=mesh,
        scratch_types=[
            pltpu.VMEM((HB, EB), jnp.int32),
            pltpu.VMEM((NB, EB), jnp.int32),
            pltpu.VMEM((EB, dc), F32),
            pltpu.VMEM((EB, dc), F32),
            pltpu.VMEM_SHARED((NP, dc), F32),
            pltpu.SemaphoreType.DMA,
            pltpu.SemaphoreType.DMA,
        ],
    )
    def k(u_hbm, idx16_hbm, dst_hbm, out_hbm, gidx_v, didx_v, rows0_v,
          rows1_v, acc_sh, sem0, sem1):
        core = lax.axis_index("c")
        s = lax.axis_index("s")
        pltpu.sync_copy(dst_hbm.at[pl.ds(s * NB, NB)], didx_v)
        for kk in range(ncpc):
            c = kk * 2 + core

            @pl.when(c < C)
            def _():
                pltpu.sync_copy(
                    u_hbm.at[pl.ds(c * NP + s * NROW, NROW)],
                    acc_sh.at[pl.ds(s * NROW, NROW)],
                )
                plsc.subcore_barrier()
                for h in range(NB // HB):
                    pltpu.sync_copy(
                        idx16_hbm.at[c, pl.ds(s * NB + h * HB, HB)], gidx_v)
                    pltpu.async_copy(u_hbm.at[gidx_v.at[0]], rows0_v, sem0)

                    def body(p, carry, h=h):
                        b0 = 2 * p
                        d0 = h * HB + b0
                        pltpu.async_copy(u_hbm.at[gidx_v.at[b0 + 1]],
                                         rows1_v, sem1)
                        pltpu.make_async_copy(u_hbm.at[gidx_v.at[b0]],
                                              rows0_v, sem0).wait()
                        pltpu.sync_copy(rows0_v, acc_sh.at[didx_v.at[d0]],
                                        add=True)

                        @pl.when(b0 + 2 < HB)
                        def _():
                            pltpu.async_copy(u_hbm.at[gidx_v.at[b0 + 2]],
                                             rows0_v, sem0)

                        pltpu.make_async_copy(u_hbm.at[gidx_v.at[b0 + 1]],
                                              rows1_v, sem1).wait()
                        pltpu.sync_copy(rows1_v,
                                        acc_sh.at[didx_v.at[d0 + 1]],
                                        add=True)
                        return carry

                    lax.fori_loop(0, HB // 2, body, 0)
                plsc.subcore_barrier()
                pltpu.sync_copy(
                    acc_sh.at[pl.ds(s * NROW, NROW)],
                    out_hbm.at[c, pl.ds(s * NROW, NROW)],
                )

    return k


def _sc_deg():
    mesh = plsc.VectorSubcoreMesh(core_axis_name="c", subcore_axis_name="s")

    @functools.partial(
        pl.kernel,
        out_type=jax.ShapeDtypeStruct((NP, 16), F32),
        mesh=mesh,
        scratch_types=[
            pltpu.VMEM((NB, EB), jnp.int32),
            pltpu.VMEM((EB, 16), F32),
            pltpu.VMEM_SHARED((NP, 16), F32),
        ],
    )
    def k(dst_hbm, zeros_hbm, out_hbm, didx_v, ones_v, acc_sh):
        core = lax.axis_index("c")
        s = lax.axis_index("s")

        @pl.when(core == 0)
        def _():
            pltpu.sync_copy(dst_hbm.at[pl.ds(s * NB, NB)], didx_v)
            pltpu.sync_copy(
                zeros_hbm.at[pl.ds(s * NROW, NROW)],
                acc_sh.at[pl.ds(s * NROW, NROW)],
            )
            vec = jnp.where(lax.iota(jnp.int32, 16) == 0, 1.0, 0.0).astype(F32)
            for r in range(EB):
                ones_v[r] = vec
            plsc.subcore_barrier()

            def body(b, carry):
                pltpu.sync_copy(ones_v, acc_sh.at[didx_v.at[b]], add=True)
                return carry

            lax.fori_loop(0, NB, body, 0)
            plsc.subcore_barrier()
            pltpu.sync_copy(
                acc_sh.at[pl.ds(s * NROW, NROW)],
                out_hbm.at[pl.ds(s * NROW, NROW)],
            )

    return k


def _agg_call(C, u_flat, idx16, dst2d, dc=DC):
    return _sc_agg(C, dc)(u_flat, idx16, dst2d)


def _deg_call(dst2d, zeros16):
    return _sc_deg()(dst2d, zeros16)


def _rowmask(i, w, bm=BM):
    rid = lax.broadcasted_iota(jnp.int32, (bm, w), 0) + i * bm
    return rid < NN


def _bn_coef(cp):
    mu = cp[1:2] * (1.0 / NN)
    var = cp[2:3] * (1.0 / NN) - mu * mu
    alpha = cp[3:4] * lax.rsqrt(var + 1e-5)
    beta = cp[4:5] - mu * alpha
    return alpha, beta


def _pre(pre, s_blk, dinv_blk, cp):
    if pre == "dinv":
        return dinv_blk * s_blk
    if pre == "scn":
        z = jnp.maximum(dinv_blk * s_blk + cp[0:1], 0.0)
        alpha, beta = _bn_coef(cp)
        return z * alpha + beta
    alpha, beta = _bn_coef(cp)
    return s_blk * alpha + beta


def _mm_rs(S, W, dinv, bo):
    C = S.shape[0]
    dout = W.shape[1]
    bm = 512
    n_i = NP // bm

    def body(s_ref, w_ref, dinv_ref, bo_ref, z_ref, st_ref, acc_ref):
        i = pl.program_id(0)
        k = pl.program_id(1)
        nk = pl.num_programs(1)

        @pl.when(k == 0)
        def _():
            acc_ref[...] = jnp.zeros((bm, dout), F32)

        lhs = dinv_ref[...] * s_ref[0]
        acc_ref[...] += jnp.dot(lhs, w_ref[...], preferred_element_type=F32)

        @pl.when(k == nk - 1)
        def _():
            z = jnp.maximum(acc_ref[...] + bo_ref[0:1], 0.0)
            z_ref[...] = z
            zm = jnp.where(_rowmask(i, dout, bm), z, 0.0)

            @pl.when(i == 0)
            def _():
                st_ref[...] = jnp.zeros((8, dout), F32)

            st_ref[1:2] += jnp.sum(zm, 0, keepdims=True)
            st_ref[2:3] += jnp.sum(zm * zm, 0, keepdims=True)

    return pl.pallas_call(
        body,
        grid=(n_i, C),
        in_specs=[
            pl.BlockSpec((1, bm, DC), lambda i, k: (k, i, 0)),
            pl.BlockSpec((DC, dout), lambda i, k: (k, 0)),
            pl.BlockSpec((bm, DC), lambda i, k: (i, 0)),
            pl.BlockSpec((8, dout), lambda i, k: (0, 0)),
        ],
        out_specs=[
            pl.BlockSpec((bm, dout), lambda i, k: (i, 0)),
            pl.BlockSpec((8, dout), lambda i, k: (0, 0)),
        ],
        out_shape=[
            jax.ShapeDtypeStruct((NP, dout), F32),
            jax.ShapeDtypeStruct((8, dout), F32),
        ],
        scratch_shapes=[pltpu.VMEM((bm, dout), F32)],
    )(S, W, dinv, bo)


def _mm_uc(lhs_in, W, dinv, cp, pre, bk, jpart=(0, 1), ocw=DC):
    chunks = pre == "scn"
    K = W.shape[0]
    jidx, nj = jpart
    dout = W.shape[1] // nj
    cout = dout // ocw
    n_i = NP // BM
    n_k = K // bk

    def body(l_ref, w_ref, dinv_ref, cp_ref, u_ref, acc_ref):
        k = pl.program_id(1)
        nk = pl.num_programs(1)

        @pl.when(k == 0)
        def _():
            acc_ref[...] = jnp.zeros((BM, dout), F32)

        s_blk = l_ref[0] if chunks else l_ref[...]
        lhs = _pre(pre, s_blk, dinv_ref[...], cp_ref[...])
        acc_ref[...] += jnp.dot(lhs, w_ref[...], preferred_element_type=F32)

        @pl.when(k == nk - 1)
        def _():
            for cc in range(cout):
                u_ref[cc] = (dinv_ref[:, :ocw]
                             * acc_ref[:, cc * ocw:(cc + 1) * ocw])

    lhs_spec = (
        pl.BlockSpec((1, BM, DC), lambda i, k: (k, i, 0))
        if chunks else pl.BlockSpec((BM, bk), lambda i, k: (i, k))
    )
    return pl.pallas_call(
        body,
        grid=(n_i, n_k),
        in_specs=[
            lhs_spec,
            pl.BlockSpec((bk, dout), lambda i, k: (k, jidx)),
            pl.BlockSpec((BM, DC), lambda i, k: (i, 0)),
            pl.BlockSpec((8, bk), lambda i, k: (0, k)),
        ],
        out_specs=[pl.BlockSpec((cout, BM, ocw), lambda i, k: (0, i, 0))],
        out_shape=[jax.ShapeDtypeStruct((cout, NP, ocw), F32)],
        scratch_shapes=[pltpu.VMEM((BM, dout), F32)],
    )(lhs_in, W, dinv, cp)[0]


def _mm_uc_part(S, W, dinv, cp, koff):
    C = S.shape[0]
    dout = W.shape[1]

    def body(s_ref, w_ref, dinv_ref, cp_ref, p_ref, acc_ref):
        k = pl.program_id(1)
        nk = pl.num_programs(1)

        @pl.when(k == 0)
        def _():
            acc_ref[...] = jnp.zeros((BM, dout), F32)

        lhs = _pre("scn", s_ref[0], dinv_ref[...], cp_ref[...])
        acc_ref[...] += jnp.dot(lhs, w_ref[...], preferred_element_type=F32)

        @pl.when(k == nk - 1)
        def _():
            p_ref[...] = acc_ref[...]

    return pl.pallas_call(
        body,
        grid=(NP // BM, C),
        in_specs=[
            pl.BlockSpec((1, BM, DC), lambda i, k: (k, i, 0)),
            pl.BlockSpec((DC, dout), lambda i, k: (k + koff, 0)),
            pl.BlockSpec((BM, DC), lambda i, k: (i, 0)),
            pl.BlockSpec((8, DC), lambda i, k: (0, k)),
        ],
        out_specs=[pl.BlockSpec((BM, dout), lambda i, k: (i, 0))],
        out_shape=[jax.ShapeDtypeStruct((NP, dout), F32)],
        scratch_shapes=[pltpu.VMEM((BM, dout), F32)],
    )(S, W, dinv, cp)[0]


def _mm_uc_fin(S, W, dinv, cp, part, koff):
    C = S.shape[0]
    dout = W.shape[1]
    cout = dout // DC

    def body(s_ref, w_ref, dinv_ref, cp_ref, p_ref, u_ref, acc_ref):
        k = pl.program_id(1)
        nk = pl.num_programs(1)

        @pl.when(k == 0)
        def _():
            acc_ref[...] = p_ref[...]

        lhs = _pre("scn", s_ref[0], dinv_ref[...], cp_ref[...])
        acc_ref[...] += jnp.dot(lhs, w_ref[...], preferred_element_type=F32)

        @pl.when(k == nk - 1)
        def _():
            for cc in range(cout):
                u_ref[cc] = dinv_ref[...] * acc_ref[:, cc * DC:(cc + 1) * DC]

    return pl.pallas_call(
        body,
        grid=(NP // BM, C),
        in_specs=[
            pl.BlockSpec((1, BM, DC), lambda i, k: (k, i, 0)),
            pl.BlockSpec((DC, dout), lambda i, k: (k + koff, 0)),
            pl.BlockSpec((BM, DC), lambda i, k: (i, 0)),
            pl.BlockSpec((8, DC), lambda i, k: (0, k)),
            pl.BlockSpec((BM, dout), lambda i, k: (i, 0)),
        ],
        out_specs=[pl.BlockSpec((cout, BM, DC), lambda i, k: (0, i, 0))],
        out_shape=[jax.ShapeDtypeStruct((cout, NP, DC), F32)],
        scratch_shapes=[pltpu.VMEM((BM, dout), F32)],
    )(S, W, dinv, cp, part)[0]


def _stats(S, dinv, cp):
    C = S.shape[0]

    def body(s_ref, dinv_ref, cp_ref, st_ref):
        i = pl.program_id(1)
        z = jnp.maximum(dinv_ref[...] * s_ref[0] + cp_ref[0:1], 0.0)
        zm = jnp.where(_rowmask(i, DC), z, 0.0)

        @pl.when(i == 0)
        def _():
            st_ref[...] = jnp.zeros((8, DC), F32)

        st_ref[1:2] += jnp.sum(zm, 0, keepdims=True)
        st_ref[2:3] += jnp.sum(zm * zm, 0, keepdims=True)

    return pl.pallas_call(
        body,
        grid=(C, NP // BM),
        in_specs=[
            pl.BlockSpec((1, BM, DC), lambda c, i: (c, i, 0)),
            pl.BlockSpec((BM, DC), lambda c, i: (i, 0)),
            pl.BlockSpec((8, DC), lambda c, i: (0, c)),
        ],
        out_specs=pl.BlockSpec((8, DC), lambda c, i: (0, c)),
        out_shape=jax.ShapeDtypeStruct((8, C * DC), F32),
    )(S, dinv, cp)


def _chunk_x(xp, dinv):
    C = xp.shape[1] // DC

    def body(x_ref, dinv_ref, u_ref):
        u_ref[0] = dinv_ref[...] * x_ref[...]

    return pl.pallas_call(
        body,
        grid=(C, NP // BM),
        in_specs=[
            pl.BlockSpec((BM, DC), lambda c, i: (i, c)),
            pl.BlockSpec((BM, DC), lambda c, i: (i, 0)),
        ],
        out_specs=pl.BlockSpec((1, BM, DC), lambda c, i: (c, i, 0)),
        out_shape=jax.ShapeDtypeStruct((C, NP, DC), F32),
    )(xp, dinv)


def _chunk_u5(S4, dinv, cp):
    C = S4.shape[0]

    def body(s_ref, dinv_ref, cp_ref, u_ref):
        h = _pre("scn", s_ref[0], dinv_ref[...], cp_ref[...])
        u_ref[0] = dinv_ref[...] * h

    return pl.pallas_call(
        body,
        grid=(C, NP // BM),
        in_specs=[
            pl.BlockSpec((1, BM, DC), lambda c, i: (c, i, 0)),
            pl.BlockSpec((BM, DC), lambda c, i: (i, 0)),
            pl.BlockSpec((8, DC), lambda c, i: (0, c)),
        ],
        out_specs=pl.BlockSpec((1, BM, DC), lambda c, i: (c, i, 0)),
        out_shape=jax.ShapeDtypeStruct((C, NP, DC), F32),
    )(S4, dinv, cp)


def _dinvk(deg16):
    def body(d_ref, o_ref):
        d = d_ref[:, 0:1] + 1.0
        m = _rowmask(pl.program_id(0), 1)
        v = jnp.where(m, lax.rsqrt(d), 0.0)
        o_ref[...] = jnp.broadcast_to(v, (BM, DC))

    return pl.pallas_call(
        body,
        grid=(NP // BM,),
        in_specs=[pl.BlockSpec((BM, 16), lambda i: (i, 0))],
        out_specs=pl.BlockSpec((BM, DC), lambda i: (i, 0)),
        out_shape=jax.ShapeDtypeStruct((NP, DC), F32),
    )(deg16)


def _final(S7, dinv, batchb, b7p):
    def body(s_ref, dinv_ref, b_ref, b7_ref, o_ref):
        i = pl.program_id(0)
        m = _rowmask(i, DC)
        w = dinv_ref[...] * s_ref[0] + jnp.where(m, b7_ref[0:1], 0.0)
        gcol = lax.broadcasted_iota(jnp.int32, (BM, DC), 1)
        oh = (b_ref[...] == gcol).astype(F32)
        p = lax.dot_general(oh, w, (((0,), (0,)), ((), ())),
                            preferred_element_type=F32)

        @pl.when(i == 0)
        def _():
            o_ref[...] = jnp.zeros((DC, DC), F32)

        o_ref[...] += p

    return pl.pallas_call(
        body,
        grid=(NP // BM,),
        in_specs=[
            pl.BlockSpec((1, BM, DC), lambda i: (0, i, 0)),
            pl.BlockSpec((BM, DC), lambda i: (i, 0)),
            pl.BlockSpec((BM, DC), lambda i: (i, 0)),
            pl.BlockSpec((8, DC), lambda i: (0, 0)),
        ],
        out_specs=pl.BlockSpec((DC, DC), lambda i: (0, 0)),
        out_shape=jax.ShapeDtypeStruct((DC, DC), F32),
    )(S7, dinv, batchb, b7p)


def _cp_pack(K, b=None, st=None, g=None, be=None):
    P = jnp.zeros((8, K), F32)
    if b is not None:
        P = P.at[0, : b.shape[0]].set(b)
    if g is not None:
        P = P.at[3, : g.shape[0]].set(g)
    if be is not None:
        P = P.at[4, : be.shape[0]].set(be)
    if st is not None:
        P = P + st
    return P


def _bo_pack(b, dout):
    return jnp.zeros((8, dout), F32).at[0, : b.shape[0]].set(b)


def kernel(x, edge_index, batch, W1, b1, W2, b2, W3, b3, W4, b4, W5, b5,
           W6, b6, W7, b7, g1, be1, g2, be2, g3, be3, g4, be4, g5, be5,
           g6, be6):
    src = edge_index[0]
    dst = edge_index[1]

    dst2d = dst.reshape(16 * NB, EB)
    offs = (jnp.arange(CMAX, dtype=jnp.int32) * NP)[:, None]
    idx16 = (src[None, :] + offs).reshape(CMAX, 16 * NB, EB)
    zeros16 = jnp.zeros((NP, 16), F32)
    batchb = jnp.broadcast_to(
        jnp.pad(batch, (0, NP - NN), constant_values=NG)[:, None], (NP, DC)
    )
    xp = jnp.pad(x, ((0, NP - NN), (0, 512 - x.shape[1])))
    W1p = jnp.pad(W1, ((0, 512 - W1.shape[0]), (0, 0)))
    W7p = jnp.pad(W7, ((0, 0), (0, DC - W7.shape[1])))
    b7p = jnp.zeros((8, DC), F32).at[0, 0].set(b7[0])

    deg16 = _deg_call(dst2d, zeros16)
    dinv = _dinvk(deg16)

    u1 = _chunk_x(xp, dinv)
    S1 = _agg_call(4, u1.reshape(4 * NP, DC), idx16, dst2d)
    z1, st1 = _mm_rs(S1, W1p, dinv, _bo_pack(b1, 4096))

    cp2 = _cp_pack(4096, st=st1, g=g1, be=be1)
    u2a = _mm_uc(z1, W2, dinv, cp2, "bn", 512, jpart=(0, 2))
    S2a = _agg_call(8, u2a.reshape(8 * NP, DC), idx16, dst2d)
    u2b = _mm_uc(z1, W2, dinv, cp2, "bn", 512, jpart=(1, 2))
    S2b = _agg_call(8, u2b.reshape(8 * NP, DC), idx16, dst2d)

    st2a = _stats(S2a, dinv, _cp_pack(1024, b=b2[:1024]))
    st2b = _stats(S2b, dinv, _cp_pack(1024, b=b2[1024:]))
    cp3a = _cp_pack(1024, b=b2[:1024], st=st2a, g=g2[:1024], be=be2[:1024])
    cp3b = _cp_pack(1024, b=b2[1024:], st=st2b, g=g2[1024:], be=be2[1024:])
    p3 = _mm_uc_part(S2a, W3, dinv, cp3a, 0)
    u3 = _mm_uc_fin(S2b, W3, dinv, cp3b, p3, 8)
    S3 = _agg_call(8, u3.reshape(8 * NP, DC), idx16, dst2d)

    st3 = _stats(S3, dinv, _cp_pack(1024, b=b3))
    u4 = _mm_uc(S3, W4, dinv, _cp_pack(1024, b=b3, st=st3, g=g3, be=be3),
                "scn", DC)
    S4 = _agg_call(8, u4.reshape(8 * NP, DC), idx16, dst2d)

    st4 = _stats(S4, dinv, _cp_pack(1024, b=b4))
    u5 = _chunk_u5(S4, dinv, _cp_pack(1024, b=b4, st=st4, g=g4, be=be4))
    S5 = _agg_call(8, u5.reshape(8 * NP, DC), idx16, dst2d)
    z5, st5 = _mm_rs(S5, W5, dinv, _bo_pack(b5, 2048))

    u6 = _mm_uc(z5, W6, dinv, _cp_pack(2048, st=st5, g=g5, be=be5), "bn", 512)
    S6 = _agg_call(2, u6.reshape(2 * NP, DC), idx16, dst2d)

    st6 = _stats(S6, dinv, _cp_pack(256, b=b6))
    u7 = _mm_uc(S6, W7p, dinv, _cp_pack(256, b=b6, st=st6, g=g6, be=be6),
                "scn", DC)
    S7 = _agg_call(1, u7.reshape(NP, DC), idx16, dst2d)

    res = _final(S7, dinv, batchb, b7p)
    return res[:NG, 0:1]


# speedup vs baseline: 1.2614x; 1.0240x over previous
import functools

import jax
import jax.numpy as jnp
from jax import lax
from jax.experimental import pallas as pl
from jax.experimental.pallas import tpu as pltpu
from jax.experimental.pallas import tpu_sc as plsc

NN = 10000
NP = 10240
NE = 160000
NG = 16
DC = 128
BM = 1024
EB = 125
NB = 80
HB = 40
CMAX = 16
NROW = NP // 16
F32 = jnp.float32


def _sc_agg(C, dc=DC):
    mesh = plsc.VectorSubcoreMesh(core_axis_name="c", subcore_axis_name="s")
    ncpc = (C + 1) // 2

    @functools.partial(
        pl.kernel,
        out_type=jax.ShapeDtypeStruct((C, NP, dc), F32),
        mesh=mesh,
        scratch_types=[
            pltpu.VMEM((HB, EB), jnp.int32),
            pltpu.VMEM((NB, EB), jnp.int32),
            pltpu.VMEM((EB, dc), F32),
            pltpu.VMEM((EB, dc), F32),
            pltpu.VMEM_SHARED((NP, dc), F32),
            pltpu.SemaphoreType.DMA,
            pltpu.SemaphoreType.DMA,
        ],
    )
    def k(u_hbm, idx16_hbm, dst_hbm, out_hbm, gidx_v, didx_v, rows0_v,
          rows1_v, acc_sh, sem0, sem1):
        core = lax.axis_index("c")
        s = lax.axis_index("s")
        pltpu.sync_copy(dst_hbm.at[pl.ds(s * NB, NB)], didx_v)
        for kk in range(ncpc):
            c = kk * 2 + core

            @pl.when(c < C)
            def _():
                pltpu.sync_copy(
                    u_hbm.at[pl.ds(c * NP + s * NROW, NROW)],
                    acc_sh.at[pl.ds(s * NROW, NROW)],
                )
                plsc.subcore_barrier()
                for h in range(NB // HB):
                    pltpu.sync_copy(
                        idx16_hbm.at[c, pl.ds(s * NB + h * HB, HB)], gidx_v)
                    pltpu.async_copy(u_hbm.at[gidx_v.at[0]], rows0_v, sem0)

                    def body(p, carry, h=h):
                        b0 = 2 * p
                        d0 = h * HB + b0
                        pltpu.async_copy(u_hbm.at[gidx_v.at[b0 + 1]],
                                         rows1_v, sem1)
                        pltpu.make_async_copy(u_hbm.at[gidx_v.at[b0]],
                                              rows0_v, sem0).wait()
                        pltpu.sync_copy(rows0_v, acc_sh.at[didx_v.at[d0]],
                                        add=True)

                        @pl.when(b0 + 2 < HB)
                        def _():
                            pltpu.async_copy(u_hbm.at[gidx_v.at[b0 + 2]],
                                             rows0_v, sem0)

                        pltpu.make_async_copy(u_hbm.at[gidx_v.at[b0 + 1]],
                                              rows1_v, sem1).wait()
                        pltpu.sync_copy(rows1_v,
                                        acc_sh.at[didx_v.at[d0 + 1]],
                                        add=True)
                        return carry

                    lax.fori_loop(0, HB // 2, body, 0)
                plsc.subcore_barrier()
                pltpu.sync_copy(
                    acc_sh.at[pl.ds(s * NROW, NROW)],
                    out_hbm.at[c, pl.ds(s * NROW, NROW)],
                )

    return k


def _sc_deg():
    mesh = plsc.VectorSubcoreMesh(core_axis_name="c", subcore_axis_name="s")

    @functools.partial(
        pl.kernel,
        out_type=jax.ShapeDtypeStruct((NP, 16), F32),
        mesh=mesh,
        scratch_types=[
            pltpu.VMEM((NB, EB), jnp.int32),
            pltpu.VMEM((EB, 16), F32),
            pltpu.VMEM_SHARED((NP, 16), F32),
        ],
    )
    def k(dst_hbm, zeros_hbm, out_hbm, didx_v, ones_v, acc_sh):
        core = lax.axis_index("c")
        s = lax.axis_index("s")

        @pl.when(core == 0)
        def _():
            pltpu.sync_copy(dst_hbm.at[pl.ds(s * NB, NB)], didx_v)
            pltpu.sync_copy(
                zeros_hbm.at[pl.ds(s * NROW, NROW)],
                acc_sh.at[pl.ds(s * NROW, NROW)],
            )
            vec = jnp.where(lax.iota(jnp.int32, 16) == 0, 1.0, 0.0).astype(F32)
            for r in range(EB):
                ones_v[r] = vec
            plsc.subcore_barrier()

            def body(b, carry):
                pltpu.sync_copy(ones_v, acc_sh.at[didx_v.at[b]], add=True)
                return carry

            lax.fori_loop(0, NB, body, 0)
            plsc.subcore_barrier()
            pltpu.sync_copy(
                acc_sh.at[pl.ds(s * NROW, NROW)],
                out_hbm.at[pl.ds(s * NROW, NROW)],
            )

    return k


def _agg_call(C, u_flat, idx16, dst2d, dc=DC):
    return _sc_agg(C, dc)(u_flat, idx16, dst2d)


def _deg_call(dst2d, zeros16):
    return _sc_deg()(dst2d, zeros16)


def _rowmask(i, w, bm=BM):
    rid = lax.broadcasted_iota(jnp.int32, (bm, w), 0) + i * bm
    return rid < NN


def _bn_coef(cp):
    mu = cp[1:2] * (1.0 / NN)
    var = cp[2:3] * (1.0 / NN) - mu * mu
    alpha = cp[3:4] * lax.rsqrt(var + 1e-5)
    beta = cp[4:5] - mu * alpha
    return alpha, beta


def _pre(pre, s_blk, dinv_blk, cp):
    if pre == "dinv":
        return dinv_blk * s_blk
    if pre == "scn":
        z = jnp.maximum(dinv_blk * s_blk + cp[0:1], 0.0)
        alpha, beta = _bn_coef(cp)
        return z * alpha + beta
    alpha, beta = _bn_coef(cp)
    return s_blk * alpha + beta


def _mm_rs(S, W, dinv, bo):
    C = S.shape[0]
    dout = W.shape[1]
    bm = 512
    n_i = NP // bm

    def body(s_ref, w_ref, dinv_ref, bo_ref, z_ref, st_ref, acc_ref):
        i = pl.program_id(0)
        k = pl.program_id(1)
        nk = pl.num_programs(1)

        @pl.when(k == 0)
        def _():
            acc_ref[...] = jnp.zeros((bm, dout), F32)

        lhs = dinv_ref[...] * s_ref[0]
        acc_ref[...] += jnp.dot(lhs, w_ref[...], preferred_element_type=F32)

        @pl.when(k == nk - 1)
        def _():
            z = jnp.maximum(acc_ref[...] + bo_ref[0:1], 0.0)
            z_ref[...] = z
            zm = jnp.where(_rowmask(i, dout, bm), z, 0.0)

            @pl.when(i == 0)
            def _():
                st_ref[...] = jnp.zeros((8, dout), F32)

            st_ref[1:2] += jnp.sum(zm, 0, keepdims=True)
            st_ref[2:3] += jnp.sum(zm * zm, 0, keepdims=True)

    return pl.pallas_call(
        body,
        grid=(n_i, C),
        in_specs=[
            pl.BlockSpec((1, bm, DC), lambda i, k: (k, i, 0)),
            pl.BlockSpec((DC, dout), lambda i, k: (k, 0)),
            pl.BlockSpec((bm, DC), lambda i, k: (i, 0)),
            pl.BlockSpec((8, dout), lambda i, k: (0, 0)),
        ],
        out_specs=[
            pl.BlockSpec((bm, dout), lambda i, k: (i, 0)),
            pl.BlockSpec((8, dout), lambda i, k: (0, 0)),
        ],
        out_shape=[
            jax.ShapeDtypeStruct((NP, dout), F32),
            jax.ShapeDtypeStruct((8, dout), F32),
        ],
        scratch_shapes=[pltpu.VMEM((bm, dout), F32)],
    )(S, W, dinv, bo)


def _mm_rs_fin(S, W, dinv, bo, part, koff):
    C = S.shape[0]
    dout = W.shape[1]
    bm = 512
    n_i = NP // bm

    def body(s_ref, w_ref, dinv_ref, bo_ref, p_ref, z_ref, st_ref, acc_ref):
        i = pl.program_id(0)
        k = pl.program_id(1)
        nk = pl.num_programs(1)

        @pl.when(k == 0)
        def _():
            acc_ref[...] = p_ref[...]

        lhs = dinv_ref[...] * s_ref[0]
        acc_ref[...] += jnp.dot(lhs, w_ref[...], preferred_element_type=F32)

        @pl.when(k == nk - 1)
        def _():
            z = jnp.maximum(acc_ref[...] + bo_ref[0:1], 0.0)
            z_ref[...] = z
            zm = jnp.where(_rowmask(i, dout, bm), z, 0.0)

            @pl.when(i == 0)
            def _():
                st_ref[...] = jnp.zeros((8, dout), F32)

            st_ref[1:2] += jnp.sum(zm, 0, keepdims=True)
            st_ref[2:3] += jnp.sum(zm * zm, 0, keepdims=True)

    return pl.pallas_call(
        body,
        grid=(n_i, C),
        in_specs=[
            pl.BlockSpec((1, bm, DC), lambda i, k: (k, i, 0)),
            pl.BlockSpec((DC, dout), lambda i, k: (k + koff, 0)),
            pl.BlockSpec((bm, DC), lambda i, k: (i, 0)),
            pl.BlockSpec((8, dout), lambda i, k: (0, 0)),
            pl.BlockSpec((bm, dout), lambda i, k: (i, 0)),
        ],
        out_specs=[
            pl.BlockSpec((bm, dout), lambda i, k: (i, 0)),
            pl.BlockSpec((8, dout), lambda i, k: (0, 0)),
        ],
        out_shape=[
            jax.ShapeDtypeStruct((NP, dout), F32),
            jax.ShapeDtypeStruct((8, dout), F32),
        ],
        scratch_shapes=[pltpu.VMEM((bm, dout), F32)],
    )(S, W, dinv, bo, part)


def _mm_uc(lhs_in, W, dinv, cp, pre, bk, jpart=(0, 1), ocw=DC):
    chunks = pre == "scn"
    K = W.shape[0]
    jidx, nj = jpart
    dout = W.shape[1] // nj
    cout = dout // ocw
    n_i = NP // BM
    n_k = K // bk

    def body(l_ref, w_ref, dinv_ref, cp_ref, u_ref, acc_ref):
        k = pl.program_id(1)
        nk = pl.num_programs(1)

        @pl.when(k == 0)
        def _():
            acc_ref[...] = jnp.zeros((BM, dout), F32)

        s_blk = l_ref[0] if chunks else l_ref[...]
        lhs = _pre(pre, s_blk, dinv_ref[...], cp_ref[...])
        acc_ref[...] += jnp.dot(lhs, w_ref[...], preferred_element_type=F32)

        @pl.when(k == nk - 1)
        def _():
            for cc in range(cout):
                u_ref[cc] = (dinv_ref[:, :ocw]
                             * acc_ref[:, cc * ocw:(cc + 1) * ocw])

    lhs_spec = (
        pl.BlockSpec((1, BM, DC), lambda i, k: (k, i, 0))
        if chunks else pl.BlockSpec((BM, bk), lambda i, k: (i, k))
    )
    return pl.pallas_call(
        body,
        grid=(n_i, n_k),
        in_specs=[
            lhs_spec,
            pl.BlockSpec((bk, dout), lambda i, k: (k, jidx)),
            pl.BlockSpec((BM, DC), lambda i, k: (i, 0)),
            pl.BlockSpec((8, bk), lambda i, k: (0, k)),
        ],
        out_specs=[pl.BlockSpec((cout, BM, ocw), lambda i, k: (0, i, 0))],
        out_shape=[jax.ShapeDtypeStruct((cout, NP, ocw), F32)],
        scratch_shapes=[pltpu.VMEM((BM, dout), F32)],
    )(lhs_in, W, dinv, cp)[0]


def _mm_uc_part(S, W, dinv, cp, koff, pre="scn"):
    C = S.shape[0]
    dout = W.shape[1]

    def body(s_ref, w_ref, dinv_ref, cp_ref, p_ref, acc_ref):
        k = pl.program_id(1)
        nk = pl.num_programs(1)

        @pl.when(k == 0)
        def _():
            acc_ref[...] = jnp.zeros((BM, dout), F32)

        lhs = _pre(pre, s_ref[0], dinv_ref[...], cp_ref[...])
        acc_ref[...] += jnp.dot(lhs, w_ref[...], preferred_element_type=F32)

        @pl.when(k == nk - 1)
        def _():
            p_ref[...] = acc_ref[...]

    return pl.pallas_call(
        body,
        grid=(NP // BM, C),
        in_specs=[
            pl.BlockSpec((1, BM, DC), lambda i, k: (k, i, 0)),
            pl.BlockSpec((DC, dout), lambda i, k: (k + koff, 0)),
            pl.BlockSpec((BM, DC), lambda i, k: (i, 0)),
            pl.BlockSpec((8, DC), lambda i, k: (0, k)),
        ],
        out_specs=[pl.BlockSpec((BM, dout), lambda i, k: (i, 0))],
        out_shape=[jax.ShapeDtypeStruct((NP, dout), F32)],
        scratch_shapes=[pltpu.VMEM((BM, dout), F32)],
    )(S, W, dinv, cp)[0]


def _mm_uc_fin(S, W, dinv, cp, part, koff):
    C = S.shape[0]
    dout = W.shape[1]
    cout = dout // DC

    def body(s_ref, w_ref, dinv_ref, cp_ref, p_ref, u_ref, acc_ref):
        k = pl.program_id(1)
        nk = pl.num_programs(1)

        @pl.when(k == 0)
        def _():
            acc_ref[...] = p_ref[...]

        lhs = _pre("scn", s_ref[0], dinv_ref[...], cp_ref[...])
        acc_ref[...] += jnp.dot(lhs, w_ref[...], preferred_element_type=F32)

        @pl.when(k == nk - 1)
        def _():
            for cc in range(cout):
                u_ref[cc] = dinv_ref[...] * acc_ref[:, cc * DC:(cc + 1) * DC]

    return pl.pallas_call(
        body,
        grid=(NP // BM, C),
        in_specs=[
            pl.BlockSpec((1, BM, DC), lambda i, k: (k, i, 0)),
            pl.BlockSpec((DC, dout), lambda i, k: (k + koff, 0)),
            pl.BlockSpec((BM, DC), lambda i, k: (i, 0)),
            pl.BlockSpec((8, DC), lambda i, k: (0, k)),
            pl.BlockSpec((BM, dout), lambda i, k: (i, 0)),
        ],
        out_specs=[pl.BlockSpec((cout, BM, DC), lambda i, k: (0, i, 0))],
        out_shape=[jax.ShapeDtypeStruct((cout, NP, DC), F32)],
        scratch_shapes=[pltpu.VMEM((BM, dout), F32)],
    )(S, W, dinv, cp, part)[0]


def _stats(S, dinv, cp):
    C = S.shape[0]

    def body(s_ref, dinv_ref, cp_ref, st_ref):
        i = pl.program_id(1)
        z = jnp.maximum(dinv_ref[...] * s_ref[0] + cp_ref[0:1], 0.0)
        zm = jnp.where(_rowmask(i, DC), z, 0.0)

        @pl.when(i == 0)
        def _():
            st_ref[...] = jnp.zeros((8, DC), F32)

        st_ref[1:2] += jnp.sum(zm, 0, keepdims=True)
        st_ref[2:3] += jnp.sum(zm * zm, 0, keepdims=True)

    return pl.pallas_call(
        body,
        grid=(C, NP // BM),
        in_specs=[
            pl.BlockSpec((1, BM, DC), lambda c, i: (c, i, 0)),
            pl.BlockSpec((BM, DC), lambda c, i: (i, 0)),
            pl.BlockSpec((8, DC), lambda c, i: (0, c)),
        ],
        out_specs=pl.BlockSpec((8, DC), lambda c, i: (0, c)),
        out_shape=jax.ShapeDtypeStruct((8, C * DC), F32),
    )(S, dinv, cp)


def _chunk_x(xp, dinv):
    C = xp.shape[1] // DC

    def body(x_ref, dinv_ref, u_ref):
        u_ref[0] = dinv_ref[...] * x_ref[...]

    return pl.pallas_call(
        body,
        grid=(C, NP // BM),
        in_specs=[
            pl.BlockSpec((BM, DC), lambda c, i: (i, c)),
            pl.BlockSpec((BM, DC), lambda c, i: (i, 0)),
        ],
        out_specs=pl.BlockSpec((1, BM, DC), lambda c, i: (c, i, 0)),
        out_shape=jax.ShapeDtypeStruct((C, NP, DC), F32),
    )(xp, dinv)


def _chunk_u5(S4, dinv, cp):
    C = S4.shape[0]

    def body(s_ref, dinv_ref, cp_ref, u_ref):
        h = _pre("scn", s_ref[0], dinv_ref[...], cp_ref[...])
        u_ref[0] = dinv_ref[...] * h

    return pl.pallas_call(
        body,
        grid=(C, NP // BM),
        in_specs=[
            pl.BlockSpec((1, BM, DC), lambda c, i: (c, i, 0)),
            pl.BlockSpec((BM, DC), lambda c, i: (i, 0)),
            pl.BlockSpec((8, DC), lambda c, i: (0, c)),
        ],
        out_specs=pl.BlockSpec((1, BM, DC), lambda c, i: (c, i, 0)),
        out_shape=jax.ShapeDtypeStruct((C, NP, DC), F32),
    )(S4, dinv, cp)


def _dinvk(deg16):
    def body(d_ref, o_ref):
        d = d_ref[:, 0:1] + 1.0
        m = _rowmask(pl.program_id(0), 1)
        v = jnp.where(m, lax.rsqrt(d), 0.0)
        o_ref[...] = jnp.broadcast_to(v, (BM, DC))

    return pl.pallas_call(
        body,
        grid=(NP // BM,),
        in_specs=[pl.BlockSpec((BM, 16), lambda i: (i, 0))],
        out_specs=pl.BlockSpec((BM, DC), lambda i: (i, 0)),
        out_shape=jax.ShapeDtypeStruct((NP, DC), F32),
    )(deg16)


def _final(S7, dinv, batchb, b7p):
    def body(s_ref, dinv_ref, b_ref, b7_ref, o_ref):
        i = pl.program_id(0)
        m = _rowmask(i, DC)
        w = dinv_ref[...] * s_ref[0] + jnp.where(m, b7_ref[0:1], 0.0)
        gcol = lax.broadcasted_iota(jnp.int32, (BM, DC), 1)
        oh = (b_ref[...] == gcol).astype(F32)
        p = lax.dot_general(oh, w, (((0,), (0,)), ((), ())),
                            preferred_element_type=F32)

        @pl.when(i == 0)
        def _():
            o_ref[...] = jnp.zeros((DC, DC), F32)

        o_ref[...] += p

    return pl.pallas_call(
        body,
        grid=(NP // BM,),
        in_specs=[
            pl.BlockSpec((1, BM, DC), lambda i: (0, i, 0)),
            pl.BlockSpec((BM, DC), lambda i: (i, 0)),
            pl.BlockSpec((BM, DC), lambda i: (i, 0)),
            pl.BlockSpec((8, DC), lambda i: (0, 0)),
        ],
        out_specs=pl.BlockSpec((DC, DC), lambda i: (0, 0)),
        out_shape=jax.ShapeDtypeStruct((DC, DC), F32),
    )(S7, dinv, batchb, b7p)


def _cp_pack(K, b=None, st=None, g=None, be=None):
    P = jnp.zeros((8, K), F32)
    if b is not None:
        P = P.at[0, : b.shape[0]].set(b)
    if g is not None:
        P = P.at[3, : g.shape[0]].set(g)
    if be is not None:
        P = P.at[4, : be.shape[0]].set(be)
    if st is not None:
        P = P + st
    return P


def _bo_pack(b, dout):
    return jnp.zeros((8, dout), F32).at[0, : b.shape[0]].set(b)


def kernel(x, edge_index, batch, W1, b1, W2, b2, W3, b3, W4, b4, W5, b5,
           W6, b6, W7, b7, g1, be1, g2, be2, g3, be3, g4, be4, g5, be5,
           g6, be6):
    src = edge_index[0]
    dst = edge_index[1]

    dst2d = dst.reshape(16 * NB, EB)
    offs = (jnp.arange(CMAX, dtype=jnp.int32) * NP)[:, None]
    idx16 = (src[None, :] + offs).reshape(CMAX, 16 * NB, EB)
    zeros16 = jnp.zeros((NP, 16), F32)
    batchb = jnp.broadcast_to(
        jnp.pad(batch, (0, NP - NN), constant_values=NG)[:, None], (NP, DC)
    )
    xp = jnp.pad(x, ((0, NP - NN), (0, 512 - x.shape[1])))
    W1p = jnp.pad(W1, ((0, 512 - W1.shape[0]), (0, 0)))
    W7p = jnp.pad(W7, ((0, 0), (0, DC - W7.shape[1])))
    b7p = jnp.zeros((8, DC), F32).at[0, 0].set(b7[0])

    deg16 = _deg_call(dst2d, zeros16)
    dinv = _dinvk(deg16)
    zcp = jnp.zeros((8, 2 * DC), F32)


    u1 = _chunk_x(xp, dinv)
    S1a = _agg_call(2, u1[:2].reshape(2 * NP, DC), idx16, dst2d)
    S1b = _agg_call(2, u1[2:].reshape(2 * NP, DC), idx16, dst2d)
    p1 = _mm_uc_part(S1a, W1p, dinv, zcp, 0, pre="dinv")
    z1, st1 = _mm_rs_fin(S1b, W1p, dinv, _bo_pack(b1, 4096), p1, 2)

    cp2 = _cp_pack(4096, st=st1, g=g1, be=be1)
    u2a = _mm_uc(z1, W2, dinv, cp2, "bn", 512, jpart=(0, 2))
    S2a = _agg_call(8, u2a.reshape(8 * NP, DC), idx16, dst2d)
    u2b = _mm_uc(z1, W2, dinv, cp2, "bn", 512, jpart=(1, 2))
    S2b = _agg_call(8, u2b.reshape(8 * NP, DC), idx16, dst2d)

    st2a = _stats(S2a, dinv, _cp_pack(1024, b=b2[:1024]))
    cp3a = _cp_pack(1024, b=b2[:1024], st=st2a, g=g2[:1024], be=be2[:1024])
    p3 = _mm_uc_part(S2a, W3, dinv, cp3a, 0)
    st2b = _stats(S2b, dinv, _cp_pack(1024, b=b2[1024:]))
    cp3b = _cp_pack(1024, b=b2[1024:], st=st2b, g=g2[1024:], be=be2[1024:])
    u3 = _mm_uc_fin(S2b, W3, dinv, cp3b, p3, 8)
    S3a = _agg_call(4, u3[:4].reshape(4 * NP, DC), idx16, dst2d)
    S3b = _agg_call(4, u3[4:].reshape(4 * NP, DC), idx16, dst2d)

    st3a = _stats(S3a, dinv, _cp_pack(512, b=b3[:512]))
    cp4a = _cp_pack(512, b=b3[:512], st=st3a, g=g3[:512], be=be3[:512])
    p4 = _mm_uc_part(S3a, W4, dinv, cp4a, 0)
    st3b = _stats(S3b, dinv, _cp_pack(512, b=b3[512:]))
    cp4b = _cp_pack(512, b=b3[512:], st=st3b, g=g3[512:], be=be3[512:])
    u4 = _mm_uc_fin(S3b, W4, dinv, cp4b, p4, 4)
    S4a = _agg_call(4, u4[:4].reshape(4 * NP, DC), idx16, dst2d)
    S4b = _agg_call(4, u4[4:].reshape(4 * NP, DC), idx16, dst2d)

    st4a = _stats(S4a, dinv, _cp_pack(512, b=b4[:512]))
    cp5a = _cp_pack(512, b=b4[:512], st=st4a, g=g4[:512], be=be4[:512])
    u5a = _chunk_u5(S4a, dinv, cp5a)
    st4b = _stats(S4b, dinv, _cp_pack(512, b=b4[512:]))
    cp5b = _cp_pack(512, b=b4[512:], st=st4b, g=g4[512:], be=be4[512:])
    u5b = _chunk_u5(S4b, dinv, cp5b)
    S5a = _agg_call(4, u5a.reshape(4 * NP, DC), idx16, dst2d)
    S5b = _agg_call(4, u5b.reshape(4 * NP, DC), idx16, dst2d)
    p5 = _mm_uc_part(S5a, W5, dinv, jnp.zeros((8, 4 * DC), F32), 0,
                     pre="dinv")
    z5, st5 = _mm_rs_fin(S5b, W5, dinv, _bo_pack(b5, 2048), p5, 4)

    u6 = _mm_uc(z5, W6, dinv, _cp_pack(2048, st=st5, g=g5, be=be5), "bn", 512)
    S6 = _agg_call(2, u6.reshape(2 * NP, DC), idx16, dst2d)

    st6 = _stats(S6, dinv, _cp_pack(256, b=b6))
    u7 = _mm_uc(S6, W7p, dinv, _cp_pack(256, b=b6, st=st6, g=g6, be=be6),
                "scn", DC)
    S7 = _agg_call(1, u7.reshape(NP, DC), idx16, dst2d)

    res = _final(S7, dinv, batchb, b7p)
    return res[:NG, 0:1]
